# baseline (speedup 1.0000x reference)
import jax
import jax.numpy as jnp
from jax import lax
from jax.experimental import pallas as pl
from jax.experimental.pallas import tpu as pltpu
from jax.experimental.pallas import tpu_sc as plsc

B, PRE, POST = 8, 2048, 2048
TAU_PLUS, TAU_MINUS = 0.02, 0.02
W_MIN, W_MAX = 0.0, 1.0
DT = 0.001

BN = 512
NW = 32
FLAT = B * PRE
CHUNK = FLAT // NW


def _tc_body(ps_ref, w_ref, sc_ref, wc_ref, nw_ref):
    w = w_ref[...]
    nw_ref[...] = jnp.clip(w, W_MIN, W_MAX)
    wc_ref[...] = jnp.zeros_like(wc_ref)
    sc_ref[...] = jnp.dot(ps_ref[...], w, preferred_element_type=jnp.float32)


def _sc_traces_body(pt_hbm, ps_hbm, qt_hbm, post_hbm, ptn_hbm, qtn_hbm,
                    a_v, b_v, sem):
    del sem
    wid = lax.axis_index("s") * 2 + lax.axis_index("c")
    base = wid * CHUNK
    decay = jnp.float32(jnp.exp(-DT / TAU_PLUS))
    for src_t, src_s, dst in ((pt_hbm, ps_hbm, ptn_hbm),
                              (qt_hbm, post_hbm, qtn_hbm)):
        pltpu.sync_copy(src_t.at[pl.ds(base, CHUNK)], a_v)
        pltpu.sync_copy(src_s.at[pl.ds(base, CHUNK)], b_v)
        for c in range(CHUNK // 16):
            sl = pl.ds(c * 16, 16)
            a_v[sl] = a_v[sl] * decay + b_v[sl]
        pltpu.sync_copy(a_v, dst.at[pl.ds(base, CHUNK)])


@jax.jit
def _run(pre_spikes, post_spikes, weights, pre_trace, post_trace):
    ptn_flat, qtn_flat = pl.kernel(
        _sc_traces_body,
        out_type=(
            jax.ShapeDtypeStruct((FLAT,), jnp.float32),
            jax.ShapeDtypeStruct((FLAT,), jnp.float32),
        ),
        mesh=plsc.VectorSubcoreMesh(core_axis_name="c", subcore_axis_name="s"),
        scratch_types=[
            pltpu.VMEM((CHUNK,), jnp.float32),
            pltpu.VMEM((CHUNK,), jnp.float32),
            pltpu.SemaphoreType.DMA,
        ],
    )(pre_trace.reshape(FLAT), pre_spikes.reshape(FLAT),
      post_trace.reshape(FLAT), post_spikes.reshape(FLAT))
    grid = (POST // BN,)
    sc, wc, nw = pl.pallas_call(
        _tc_body,
        grid=grid,
        in_specs=[
            pl.BlockSpec((B, PRE), lambda j: (0, 0)),
            pl.BlockSpec((PRE, BN), lambda j: (0, j)),
        ],
        out_specs=[
            pl.BlockSpec((B, BN), lambda j: (0, j)),
            pl.BlockSpec((PRE, BN), lambda j: (0, j)),
            pl.BlockSpec((PRE, BN), lambda j: (0, j)),
        ],
        out_shape=[
            jax.ShapeDtypeStruct((B, POST), jnp.float32),
            jax.ShapeDtypeStruct((PRE, POST), jnp.float32),
            jax.ShapeDtypeStruct((PRE, POST), jnp.float32),
        ],
        compiler_params=pltpu.CompilerParams(
            dimension_semantics=("parallel",),
        ),
    )(pre_spikes, weights)
    return sc, wc, ptn_flat.reshape(B, PRE), qtn_flat.reshape(B, POST), nw


def kernel(pre_spikes, post_spikes, weights, pre_trace, post_trace,
           last_pre_spike, last_post_spike, current_time):
    del last_pre_spike, last_post_spike, current_time
    sc, wc, ptn, qtn, nw = _run(pre_spikes, post_spikes, weights,
                                pre_trace, post_trace)
    return (sc, wc, ptn, qtn, nw)


# speedup vs baseline: 2.4136x; 2.4136x over previous
import functools

import jax
import jax.numpy as jnp
from jax.experimental import pallas as pl
from jax.experimental.pallas import tpu as pltpu

B, PRE, POST = 8, 2048, 2048
TAU_PLUS, TAU_MINUS = 0.02, 0.02
W_MIN, W_MAX = 0.0, 1.0
DT = 0.001

BN = 512


def _body(ps_ref, post_ref, w_ref, pt_ref, qt_ref,
          sc_ref, wc_ref, ptn_ref, qtn_ref, nw_ref):
    w = w_ref[...]
    nw_ref[...] = jnp.clip(w, W_MIN, W_MAX)
    wc_ref[...] = jnp.zeros_like(wc_ref)
    ptn_ref[...] = pt_ref[...] * jnp.float32(jnp.exp(-DT / TAU_PLUS)) + ps_ref[...]
    qtn_ref[...] = qt_ref[...] * jnp.float32(jnp.exp(-DT / TAU_MINUS)) + post_ref[...]
    sc_ref[...] = jnp.dot(ps_ref[...], w, preferred_element_type=jnp.float32)


@jax.jit
def _run(pre_spikes, post_spikes, weights, pre_trace, post_trace):
    grid = (POST // BN,)
    return pl.pallas_call(
        _body,
        grid=grid,
        in_specs=[
            pl.BlockSpec((B, PRE), lambda j: (0, 0)),
            pl.BlockSpec((B, BN), lambda j: (0, j)),
            pl.BlockSpec((PRE, BN), lambda j: (0, j)),
            pl.BlockSpec((B, PRE), lambda j: (0, 0)),
            pl.BlockSpec((B, BN), lambda j: (0, j)),
        ],
        out_specs=[
            pl.BlockSpec((B, BN), lambda j: (0, j)),
            pl.BlockSpec((PRE, BN), lambda j: (0, j)),
            pl.BlockSpec((B, PRE), lambda j: (0, 0)),
            pl.BlockSpec((B, BN), lambda j: (0, j)),
            pl.BlockSpec((PRE, BN), lambda j: (0, j)),
        ],
        out_shape=[
            jax.ShapeDtypeStruct((B, POST), jnp.float32),
            jax.ShapeDtypeStruct((PRE, POST), jnp.float32),
            jax.ShapeDtypeStruct((B, PRE), jnp.float32),
            jax.ShapeDtypeStruct((B, POST), jnp.float32),
            jax.ShapeDtypeStruct((PRE, POST), jnp.float32),
        ],
        compiler_params=pltpu.CompilerParams(
            dimension_semantics=("parallel",),
        ),
    )(pre_spikes, post_spikes, weights, pre_trace, post_trace)


def kernel(pre_spikes, post_spikes, weights, pre_trace, post_trace,
           last_pre_spike, last_post_spike, current_time):
    del last_pre_spike, last_post_spike, current_time
    sc, wc, ptn, qtn, nw = _run(pre_spikes, post_spikes, weights,
                                pre_trace, post_trace)
    return (sc, wc, ptn, qtn, nw)
